# baseline (speedup 1.0000x reference)
import functools

import jax
import jax.numpy as jnp
from jax import lax
from jax.experimental import pallas as pl
from jax.experimental.pallas import tpu as pltpu
from jax.experimental.pallas import tpu_sc as plsc

N_NODES = 10000
N_EDGES = 320000
D = 128
D_OUT = 16
G = 16

NW = 32
CH = 128
NPAD = 10240
EPAD = 327680
CPT = EPAD // NW // CH
WIN = 8
SC_C0 = 136
ZROWS = NPAD // 16

def _deg_body(dst2_hbm, out_hbm, dst_v, ones_v, z_v, deg_sh, sem):
    del sem
    cid = lax.axis_index("c")
    sid = lax.axis_index("s")
    wid = sid * 2 + cid

    def fill_ones(i, _):
        ones_v[pl.ds(i * 16, 16)] = jnp.ones((16,), jnp.float32)
        return 0

    lax.fori_loop(0, CH // 16, fill_ones, 0)

    def fill_zero(i, _):
        z_v[pl.ds(i * 16, 16)] = jnp.zeros((16,), jnp.float32)
        return 0

    lax.fori_loop(0, ZROWS // 16, fill_zero, 0)

    pltpu.sync_copy(z_v, deg_sh.at[pl.ds(sid * ZROWS, ZROWS)])
    plsc.subcore_barrier()

    pltpu.sync_copy(dst2_hbm.at[pl.ds(wid * CPT, CPT)], dst_v)

    def body(j, _):
        pltpu.sync_copy(ones_v, deg_sh.at[dst_v.at[j]], add=True)
        return 0

    lax.fori_loop(0, CPT, body, 0)
    plsc.subcore_barrier()
    pltpu.sync_copy(deg_sh.at[pl.ds(sid * ZROWS, ZROWS)],
                    out_hbm.at[cid, pl.ds(sid * ZROWS, ZROWS)])


@functools.cache
def _deg_call():
    return pl.kernel(
        _deg_body,
        out_type=jax.ShapeDtypeStruct((2, NPAD), jnp.float32),
        mesh=plsc.VectorSubcoreMesh(core_axis_name="c", subcore_axis_name="s"),
        scratch_types=[
            pltpu.VMEM((CPT, CH), jnp.int32),
            pltpu.VMEM((CH,), jnp.float32),
            pltpu.VMEM((ZROWS,), jnp.float32),
            pltpu.VMEM_SHARED((NPAD,), jnp.float32),
            pltpu.SemaphoreType.DMA,
        ],
    )


def _scatter_body(c0_chunks, y_hbm, src2_hbm, dst2_hbm, out_hbm,
                  src_v, dst_v, rows0, rows1, acc_sh, sem0, sem1):
    c1_chunks = 2 * CPT - c0_chunks
    cid = lax.axis_index("c")
    sid = lax.axis_index("s")
    bufs = (rows0, rows1)
    sems = (sem0, sem1)

    def fill_zero(i, _):
        for j in range(D // 16):
            rows0[i, pl.ds(j * 16, 16)] = jnp.zeros((16,), jnp.float32)
        return 0

    lax.fori_loop(0, CH, fill_zero, 0)
    for k in range(ZROWS // CH):
        pltpu.sync_copy(rows0, acc_sh.at[pl.ds(sid * ZROWS + k * CH, CH)])
    plsc.subcore_barrier()

    def window(w, start_chunk):
        base = start_chunk + w * WIN
        pltpu.sync_copy(src2_hbm.at[pl.ds(base, WIN)], src_v)
        pltpu.sync_copy(dst2_hbm.at[pl.ds(base, WIN)], dst_v)
        descs = [None] * WIN
        descs[0] = pltpu.async_copy(y_hbm.at[src_v.at[0]], bufs[0], sems[0])
        for b in range(WIN):
            descs[b].wait()
            if b + 1 < WIN:
                descs[b + 1] = pltpu.async_copy(
                    y_hbm.at[src_v.at[b + 1]], bufs[(b + 1) % 2],
                    sems[(b + 1) % 2])
            pltpu.sync_copy(bufs[b % 2], acc_sh.at[dst_v.at[b]], add=True)
        return start_chunk

    def run_edges(start_chunk, nchunks):
        lax.fori_loop(0, nchunks // WIN, window, start_chunk)

    @pl.when(cid == 0)
    def _():
        run_edges(sid * c0_chunks, c0_chunks)

    @pl.when(cid == 1)
    def _():
        run_edges(16 * c0_chunks + sid * c1_chunks, c1_chunks)

    plsc.subcore_barrier()
    pltpu.sync_copy(acc_sh.at[pl.ds(sid * ZROWS, ZROWS)],
                    out_hbm.at[cid, pl.ds(sid * ZROWS, ZROWS)])


@functools.cache
def _scatter_call(c0_chunks):
    return pl.kernel(
        functools.partial(_scatter_body, c0_chunks),
        out_type=jax.ShapeDtypeStruct((2, NPAD, D), jnp.float32),
        mesh=plsc.VectorSubcoreMesh(core_axis_name="c", subcore_axis_name="s"),
        scratch_types=[
            pltpu.VMEM((WIN, CH), jnp.int32),
            pltpu.VMEM((WIN, CH), jnp.int32),
            pltpu.VMEM((CH, D), jnp.float32),
            pltpu.VMEM((CH, D), jnp.float32),
            pltpu.VMEM_SHARED((NPAD, D), jnp.float32),
            pltpu.SemaphoreType.DMA,
            pltpu.SemaphoreType.DMA,
        ],
    )


def _stage_a_body(x_ref, w1_ref, degp_ref, xw_ref, y_ref, dinv_ref):
    deg = degp_ref[0, :N_NODES, :] + degp_ref[1, :N_NODES, :] + 1.0
    dinv = lax.rsqrt(deg)
    xw = jnp.dot(x_ref[:, :], w1_ref[:, :], preferred_element_type=jnp.float32)
    xw_ref[:, :] = xw
    y_ref[:, :] = xw * dinv
    dinv_ref[:, :] = dinv


_stage_a = pl.pallas_call(
    _stage_a_body,
    out_shape=[
        jax.ShapeDtypeStruct((N_NODES, D), jnp.float32),
        jax.ShapeDtypeStruct((N_NODES, D), jnp.float32),
        jax.ShapeDtypeStruct((N_NODES, 1), jnp.float32),
    ],
)


def _bn_relu(h, g_ref, be_ref):
    m = jnp.mean(h, axis=0, keepdims=True)
    v = jnp.mean((h - m) ** 2, axis=0, keepdims=True)
    h = (h - m) * lax.rsqrt(v + 1e-5) * g_ref[:, :] + be_ref[:, :]
    return jnp.maximum(h, 0.0)


def _stage_b_body(accp_ref, xw_ref, dinv_ref, b_ref, g_ref, be_ref, w2_ref,
                  xw2_ref, y2_ref):
    dinv = dinv_ref[:, :]
    acc = accp_ref[0, :N_NODES, :] + accp_ref[1, :N_NODES, :]
    h = dinv * acc + (dinv * dinv) * xw_ref[:, :] + b_ref[:, :]
    h = _bn_relu(h, g_ref, be_ref)
    xw2 = jnp.dot(h, w2_ref[:, :], preferred_element_type=jnp.float32)
    xw2_ref[:, :] = xw2
    y2_ref[:, :] = xw2 * dinv


_stage_b = pl.pallas_call(
    _stage_b_body,
    out_shape=[
        jax.ShapeDtypeStruct((N_NODES, D), jnp.float32),
        jax.ShapeDtypeStruct((N_NODES, D), jnp.float32),
    ],
)


def _stage_c_body(accp_ref, xw2_ref, dinv_ref, b_ref, g_ref, be_ref,
                  batch_ref, fc1w_ref, fc1b_ref, fc2w_ref, fc2b_ref, out_ref):
    dinv = dinv_ref[:, :]
    acc = accp_ref[0, :N_NODES, :] + accp_ref[1, :N_NODES, :]
    h = dinv * acc + (dinv * dinv) * xw2_ref[:, :] + b_ref[:, :]
    h = _bn_relu(h, g_ref, be_ref)
    gids = lax.broadcasted_iota(jnp.int32, (G, N_NODES), 0)
    onehot = (gids == batch_ref[:, :]).astype(jnp.float32)
    sums = jnp.dot(onehot, h, preferred_element_type=jnp.float32)
    counts = jnp.sum(onehot, axis=1, keepdims=True)
    pooled = sums / jnp.maximum(counts, 1.0)
    h3 = jnp.dot(pooled, fc1w_ref[:, :], preferred_element_type=jnp.float32)
    h3 = jnp.maximum(h3 + fc1b_ref[:, :], 0.0)
    logits = jnp.dot(h3, fc2w_ref[:, :], preferred_element_type=jnp.float32)
    logits = logits + fc2b_ref[:, :]
    z = logits - jnp.max(logits, axis=1, keepdims=True)
    e = jnp.exp(z)
    out_ref[:, :] = e / jnp.sum(e, axis=1, keepdims=True)


_stage_c = pl.pallas_call(
    _stage_c_body,
    out_shape=jax.ShapeDtypeStruct((G, D_OUT), jnp.float32),
)


@jax.jit
def kernel(x, edge_index, batch, W1, b1, g1, be1, W2, b2, g2, be2,
           fc1_w, fc1_b, fc2_w, fc2_b):
    src = edge_index[0].astype(jnp.int32)
    dst = edge_index[1].astype(jnp.int32)
    npad = EPAD - N_EDGES
    src2 = jnp.concatenate([src, jnp.zeros((npad,), jnp.int32)])
    src2 = src2.reshape(EPAD // CH, CH)
    dst2 = jnp.concatenate([dst, jnp.full((npad,), N_NODES, jnp.int32)])
    dst2 = dst2.reshape(EPAD // CH, CH)

    degp = _deg_call()(dst2).reshape(2, NPAD, 1)
    xw1, y1, dinv = _stage_a(x, W1, degp)
    acc1 = _scatter_call(SC_C0)(y1, src2, dst2)
    xw2, y2 = _stage_b(acc1, xw1, dinv, b1.reshape(1, D), g1.reshape(1, D),
                       be1.reshape(1, D), W2)
    acc2 = _scatter_call(SC_C0)(y2, src2, dst2)
    out = _stage_c(acc2, xw2, dinv, b2.reshape(1, D), g2.reshape(1, D),
                   be2.reshape(1, D), batch.reshape(1, N_NODES).astype(jnp.int32),
                   fc1_w, fc1_b.reshape(1, D), fc2_w, fc2_b.reshape(1, D_OUT))
    return out


# speedup vs baseline: 1.0531x; 1.0531x over previous
import functools

import jax
import jax.numpy as jnp
from jax import lax
from jax.experimental import pallas as pl
from jax.experimental.pallas import tpu as pltpu
from jax.experimental.pallas import tpu_sc as plsc

N_NODES = 10000
N_EDGES = 320000
D = 128
D_OUT = 16
G = 16

NW = 32
CH = 128
NPAD = 10240
EPAD = 327680
CPT = EPAD // NW // CH
WIN = 8
SC_C0 = 152
ZROWS = NPAD // 16

def _deg_body(dst2_hbm, out_hbm, dst_v, ones_v, z_v, deg_sh, sem):
    del sem
    cid = lax.axis_index("c")
    sid = lax.axis_index("s")
    wid = sid * 2 + cid

    def fill_ones(i, _):
        ones_v[pl.ds(i * 16, 16)] = jnp.ones((16,), jnp.float32)
        return 0

    lax.fori_loop(0, CH // 16, fill_ones, 0)

    def fill_zero(i, _):
        z_v[pl.ds(i * 16, 16)] = jnp.zeros((16,), jnp.float32)
        return 0

    lax.fori_loop(0, ZROWS // 16, fill_zero, 0)

    pltpu.sync_copy(z_v, deg_sh.at[pl.ds(sid * ZROWS, ZROWS)])
    plsc.subcore_barrier()

    pltpu.sync_copy(dst2_hbm.at[pl.ds(wid * CPT, CPT)], dst_v)

    def body(j, _):
        pltpu.sync_copy(ones_v, deg_sh.at[dst_v.at[j]], add=True)
        return 0

    lax.fori_loop(0, CPT, body, 0)
    plsc.subcore_barrier()
    pltpu.sync_copy(deg_sh.at[pl.ds(sid * ZROWS, ZROWS)],
                    out_hbm.at[cid, pl.ds(sid * ZROWS, ZROWS)])


@functools.cache
def _deg_call():
    return pl.kernel(
        _deg_body,
        out_type=jax.ShapeDtypeStruct((2, NPAD), jnp.float32),
        mesh=plsc.VectorSubcoreMesh(core_axis_name="c", subcore_axis_name="s"),
        scratch_types=[
            pltpu.VMEM((CPT, CH), jnp.int32),
            pltpu.VMEM((CH,), jnp.float32),
            pltpu.VMEM((ZROWS,), jnp.float32),
            pltpu.VMEM_SHARED((NPAD,), jnp.float32),
            pltpu.SemaphoreType.DMA,
        ],
    )


def _scatter_body(c0_chunks, y_hbm, src2_hbm, dst2_hbm, out_hbm,
                  src_v, dst_v, rows0, rows1, acc_sh, sem0, sem1):
    c1_chunks = 2 * CPT - c0_chunks
    cid = lax.axis_index("c")
    sid = lax.axis_index("s")
    bufs = (rows0, rows1)
    sems = (sem0, sem1)

    def fill_zero(i, _):
        for j in range(D // 16):
            rows0[i, pl.ds(j * 16, 16)] = jnp.zeros((16,), jnp.float32)
        return 0

    lax.fori_loop(0, CH, fill_zero, 0)
    for k in range(ZROWS // CH):
        pltpu.sync_copy(rows0, acc_sh.at[pl.ds(sid * ZROWS + k * CH, CH)])
    plsc.subcore_barrier()

    def window(w, start_chunk):
        base = start_chunk + w * WIN
        pltpu.sync_copy(src2_hbm.at[pl.ds(base, WIN)], src_v)
        pltpu.sync_copy(dst2_hbm.at[pl.ds(base, WIN)], dst_v)
        descs = [None] * WIN
        descs[0] = pltpu.async_copy(y_hbm.at[src_v.at[0]], bufs[0], sems[0])
        for b in range(WIN):
            descs[b].wait()
            if b + 1 < WIN:
                descs[b + 1] = pltpu.async_copy(
                    y_hbm.at[src_v.at[b + 1]], bufs[(b + 1) % 2],
                    sems[(b + 1) % 2])
            pltpu.sync_copy(bufs[b % 2], acc_sh.at[dst_v.at[b]], add=True)
        return start_chunk

    def run_edges(start_chunk, nchunks):
        lax.fori_loop(0, nchunks // WIN, window, start_chunk)

    @pl.when(cid == 0)
    def _():
        run_edges(sid * c0_chunks, c0_chunks)

    @pl.when(cid == 1)
    def _():
        run_edges(16 * c0_chunks + sid * c1_chunks, c1_chunks)

    plsc.subcore_barrier()
    pltpu.sync_copy(acc_sh.at[pl.ds(sid * ZROWS, ZROWS)],
                    out_hbm.at[cid, pl.ds(sid * ZROWS, ZROWS)])


@functools.cache
def _scatter_call(c0_chunks):
    return pl.kernel(
        functools.partial(_scatter_body, c0_chunks),
        out_type=jax.ShapeDtypeStruct((2, NPAD, D), jnp.float32),
        mesh=plsc.VectorSubcoreMesh(core_axis_name="c", subcore_axis_name="s"),
        scratch_types=[
            pltpu.VMEM((WIN, CH), jnp.int32),
            pltpu.VMEM((WIN, CH), jnp.int32),
            pltpu.VMEM((CH, D), jnp.float32),
            pltpu.VMEM((CH, D), jnp.float32),
            pltpu.VMEM_SHARED((NPAD, D), jnp.float32),
            pltpu.SemaphoreType.DMA,
            pltpu.SemaphoreType.DMA,
        ],
    )


def _stage_a_body(x_ref, w1_ref, degp_ref, xw_ref, y_ref, dinv_ref):
    deg = degp_ref[0, :N_NODES, :] + degp_ref[1, :N_NODES, :] + 1.0
    dinv = lax.rsqrt(deg)
    xw = jnp.dot(x_ref[:, :], w1_ref[:, :], preferred_element_type=jnp.float32)
    xw_ref[:, :] = xw
    y_ref[:, :] = xw * dinv
    dinv_ref[:, :] = dinv


_stage_a = pl.pallas_call(
    _stage_a_body,
    out_shape=[
        jax.ShapeDtypeStruct((N_NODES, D), jnp.float32),
        jax.ShapeDtypeStruct((N_NODES, D), jnp.float32),
        jax.ShapeDtypeStruct((N_NODES, 1), jnp.float32),
    ],
)


def _bn_relu(h, g_ref, be_ref):
    m = jnp.mean(h, axis=0, keepdims=True)
    v = jnp.mean((h - m) ** 2, axis=0, keepdims=True)
    h = (h - m) * lax.rsqrt(v + 1e-5) * g_ref[:, :] + be_ref[:, :]
    return jnp.maximum(h, 0.0)


def _stage_b_body(accp_ref, xw_ref, dinv_ref, b_ref, g_ref, be_ref, w2_ref,
                  xw2_ref, y2_ref):
    dinv = dinv_ref[:, :]
    acc = accp_ref[0, :N_NODES, :] + accp_ref[1, :N_NODES, :]
    h = dinv * acc + (dinv * dinv) * xw_ref[:, :] + b_ref[:, :]
    h = _bn_relu(h, g_ref, be_ref)
    xw2 = jnp.dot(h, w2_ref[:, :], preferred_element_type=jnp.float32)
    xw2_ref[:, :] = xw2
    y2_ref[:, :] = xw2 * dinv


_stage_b = pl.pallas_call(
    _stage_b_body,
    out_shape=[
        jax.ShapeDtypeStruct((N_NODES, D), jnp.float32),
        jax.ShapeDtypeStruct((N_NODES, D), jnp.float32),
    ],
)


def _stage_c_body(accp_ref, xw2_ref, dinv_ref, b_ref, g_ref, be_ref,
                  batch_ref, fc1w_ref, fc1b_ref, fc2w_ref, fc2b_ref, out_ref):
    dinv = dinv_ref[:, :]
    acc = accp_ref[0, :N_NODES, :] + accp_ref[1, :N_NODES, :]
    h = dinv * acc + (dinv * dinv) * xw2_ref[:, :] + b_ref[:, :]
    h = _bn_relu(h, g_ref, be_ref)
    gids = lax.broadcasted_iota(jnp.int32, (G, N_NODES), 0)
    onehot = (gids == batch_ref[:, :]).astype(jnp.float32)
    sums = jnp.dot(onehot, h, preferred_element_type=jnp.float32)
    counts = jnp.sum(onehot, axis=1, keepdims=True)
    pooled = sums / jnp.maximum(counts, 1.0)
    h3 = jnp.dot(pooled, fc1w_ref[:, :], preferred_element_type=jnp.float32)
    h3 = jnp.maximum(h3 + fc1b_ref[:, :], 0.0)
    logits = jnp.dot(h3, fc2w_ref[:, :], preferred_element_type=jnp.float32)
    logits = logits + fc2b_ref[:, :]
    z = logits - jnp.max(logits, axis=1, keepdims=True)
    e = jnp.exp(z)
    out_ref[:, :] = e / jnp.sum(e, axis=1, keepdims=True)


_stage_c = pl.pallas_call(
    _stage_c_body,
    out_shape=jax.ShapeDtypeStruct((G, D_OUT), jnp.float32),
)


@jax.jit
def kernel(x, edge_index, batch, W1, b1, g1, be1, W2, b2, g2, be2,
           fc1_w, fc1_b, fc2_w, fc2_b):
    src = edge_index[0].astype(jnp.int32)
    dst = edge_index[1].astype(jnp.int32)
    npad = EPAD - N_EDGES
    src2 = jnp.concatenate([src, jnp.zeros((npad,), jnp.int32)])
    src2 = src2.reshape(EPAD // CH, CH)
    dst2 = jnp.concatenate([dst, jnp.full((npad,), N_NODES, jnp.int32)])
    dst2 = dst2.reshape(EPAD // CH, CH)

    degp = _deg_call()(dst2).reshape(2, NPAD, 1)
    xw1, y1, dinv = _stage_a(x, W1, degp)
    acc1 = _scatter_call(SC_C0)(y1, src2, dst2)
    xw2, y2 = _stage_b(acc1, xw1, dinv, b1.reshape(1, D), g1.reshape(1, D),
                       be1.reshape(1, D), W2)
    acc2 = _scatter_call(SC_C0)(y2, src2, dst2)
    out = _stage_c(acc2, xw2, dinv, b2.reshape(1, D), g2.reshape(1, D),
                   be2.reshape(1, D), batch.reshape(1, N_NODES).astype(jnp.int32),
                   fc1_w, fc1_b.reshape(1, D), fc2_w, fc2_b.reshape(1, D_OUT))
    return out


# speedup vs baseline: 1.0920x; 1.0370x over previous
import functools

import jax
import jax.numpy as jnp
from jax import lax
from jax.experimental import pallas as pl
from jax.experimental.pallas import tpu as pltpu
from jax.experimental.pallas import tpu_sc as plsc

N_NODES = 10000
N_EDGES = 320000
D = 128
D_OUT = 16
G = 16

NW = 32
CH = 128
NPAD = 10240
EPAD = 327680
CPT = EPAD // NW // CH
WIN = 8
SC_C0 = 144
ZROWS = NPAD // 16

def _deg_body(dst2_hbm, out_hbm, dst_v, ones_v, z_v, deg_sh, sem):
    del sem
    cid = lax.axis_index("c")
    sid = lax.axis_index("s")
    wid = sid * 2 + cid

    def fill_ones(i, _):
        ones_v[pl.ds(i * 16, 16)] = jnp.ones((16,), jnp.float32)
        return 0

    lax.fori_loop(0, CH // 16, fill_ones, 0)

    def fill_zero(i, _):
        z_v[pl.ds(i * 16, 16)] = jnp.zeros((16,), jnp.float32)
        return 0

    lax.fori_loop(0, ZROWS // 16, fill_zero, 0)

    pltpu.sync_copy(z_v, deg_sh.at[pl.ds(sid * ZROWS, ZROWS)])
    plsc.subcore_barrier()

    pltpu.sync_copy(dst2_hbm.at[pl.ds(wid * CPT, CPT)], dst_v)

    def body(j, _):
        pltpu.sync_copy(ones_v, deg_sh.at[dst_v.at[j]], add=True)
        return 0

    lax.fori_loop(0, CPT, body, 0)
    plsc.subcore_barrier()
    pltpu.sync_copy(deg_sh.at[pl.ds(sid * ZROWS, ZROWS)],
                    out_hbm.at[cid, pl.ds(sid * ZROWS, ZROWS)])


@functools.cache
def _deg_call():
    return pl.kernel(
        _deg_body,
        out_type=jax.ShapeDtypeStruct((2, NPAD), jnp.float32),
        mesh=plsc.VectorSubcoreMesh(core_axis_name="c", subcore_axis_name="s"),
        scratch_types=[
            pltpu.VMEM((CPT, CH), jnp.int32),
            pltpu.VMEM((CH,), jnp.float32),
            pltpu.VMEM((ZROWS,), jnp.float32),
            pltpu.VMEM_SHARED((NPAD,), jnp.float32),
            pltpu.SemaphoreType.DMA,
        ],
    )


def _scatter_body(c0_chunks, y_hbm, src2_hbm, dst2_hbm, out_hbm,
                  src_v, dst_v, rows0, rows1, acc_sh, sem0, sem1):
    c1_chunks = 2 * CPT - c0_chunks
    cid = lax.axis_index("c")
    sid = lax.axis_index("s")
    bufs = (rows0, rows1)
    sems = (sem0, sem1)

    def fill_zero(i, _):
        for j in range(D // 16):
            rows0[i, pl.ds(j * 16, 16)] = jnp.zeros((16,), jnp.float32)
        return 0

    lax.fori_loop(0, CH, fill_zero, 0)
    for k in range(ZROWS // CH):
        pltpu.sync_copy(rows0, acc_sh.at[pl.ds(sid * ZROWS + k * CH, CH)])
    plsc.subcore_barrier()

    def window(w, start_chunk):
        base = start_chunk + w * WIN
        pltpu.sync_copy(src2_hbm.at[pl.ds(base, WIN)], src_v)
        pltpu.sync_copy(dst2_hbm.at[pl.ds(base, WIN)], dst_v)
        descs = [None] * WIN
        descs[0] = pltpu.async_copy(y_hbm.at[src_v.at[0]], bufs[0], sems[0])
        for b in range(WIN):
            descs[b].wait()
            if b + 1 < WIN:
                descs[b + 1] = pltpu.async_copy(
                    y_hbm.at[src_v.at[b + 1]], bufs[(b + 1) % 2],
                    sems[(b + 1) % 2])
            pltpu.sync_copy(bufs[b % 2], acc_sh.at[dst_v.at[b]], add=True)
        return start_chunk

    def run_edges(start_chunk, nchunks):
        lax.fori_loop(0, nchunks // WIN, window, start_chunk)

    @pl.when(cid == 0)
    def _():
        run_edges(sid * c0_chunks, c0_chunks)

    @pl.when(cid == 1)
    def _():
        run_edges(16 * c0_chunks + sid * c1_chunks, c1_chunks)

    plsc.subcore_barrier()
    pltpu.sync_copy(acc_sh.at[pl.ds(sid * ZROWS, ZROWS)],
                    out_hbm.at[cid, pl.ds(sid * ZROWS, ZROWS)])


@functools.cache
def _scatter_call(c0_chunks):
    return pl.kernel(
        functools.partial(_scatter_body, c0_chunks),
        out_type=jax.ShapeDtypeStruct((2, NPAD, D), jnp.float32),
        mesh=plsc.VectorSubcoreMesh(core_axis_name="c", subcore_axis_name="s"),
        scratch_types=[
            pltpu.VMEM((WIN, CH), jnp.int32),
            pltpu.VMEM((WIN, CH), jnp.int32),
            pltpu.VMEM((CH, D), jnp.float32),
            pltpu.VMEM((CH, D), jnp.float32),
            pltpu.VMEM_SHARED((NPAD, D), jnp.float32),
            pltpu.SemaphoreType.DMA,
            pltpu.SemaphoreType.DMA,
        ],
    )


def _stage_a_body(x_ref, w1_ref, degp_ref, xw_ref, y_ref, dinv_ref):
    deg = degp_ref[0, :N_NODES, :] + degp_ref[1, :N_NODES, :] + 1.0
    dinv = lax.rsqrt(deg)
    xw = jnp.dot(x_ref[:, :], w1_ref[:, :], preferred_element_type=jnp.float32)
    xw_ref[:, :] = xw
    y_ref[:, :] = xw * dinv
    dinv_ref[:, :] = dinv


_stage_a = pl.pallas_call(
    _stage_a_body,
    out_shape=[
        jax.ShapeDtypeStruct((N_NODES, D), jnp.float32),
        jax.ShapeDtypeStruct((N_NODES, D), jnp.float32),
        jax.ShapeDtypeStruct((N_NODES, 1), jnp.float32),
    ],
)


def _bn_relu(h, g_ref, be_ref):
    m = jnp.mean(h, axis=0, keepdims=True)
    v = jnp.mean((h - m) ** 2, axis=0, keepdims=True)
    h = (h - m) * lax.rsqrt(v + 1e-5) * g_ref[:, :] + be_ref[:, :]
    return jnp.maximum(h, 0.0)


def _stage_b_body(accp_ref, xw_ref, dinv_ref, b_ref, g_ref, be_ref, w2_ref,
                  xw2_ref, y2_ref):
    dinv = dinv_ref[:, :]
    acc = accp_ref[0, :N_NODES, :] + accp_ref[1, :N_NODES, :]
    h = dinv * acc + (dinv * dinv) * xw_ref[:, :] + b_ref[:, :]
    h = _bn_relu(h, g_ref, be_ref)
    xw2 = jnp.dot(h, w2_ref[:, :], preferred_element_type=jnp.float32)
    xw2_ref[:, :] = xw2
    y2_ref[:, :] = xw2 * dinv


_stage_b = pl.pallas_call(
    _stage_b_body,
    out_shape=[
        jax.ShapeDtypeStruct((N_NODES, D), jnp.float32),
        jax.ShapeDtypeStruct((N_NODES, D), jnp.float32),
    ],
)


def _stage_c_body(accp_ref, xw2_ref, dinv_ref, b_ref, g_ref, be_ref,
                  batch_ref, fc1w_ref, fc1b_ref, fc2w_ref, fc2b_ref, out_ref):
    dinv = dinv_ref[:, :]
    acc = accp_ref[0, :N_NODES, :] + accp_ref[1, :N_NODES, :]
    h = dinv * acc + (dinv * dinv) * xw2_ref[:, :] + b_ref[:, :]
    h = _bn_relu(h, g_ref, be_ref)
    gids = lax.broadcasted_iota(jnp.int32, (G, N_NODES), 0)
    onehot = (gids == batch_ref[:, :]).astype(jnp.float32)
    sums = jnp.dot(onehot, h, preferred_element_type=jnp.float32)
    counts = jnp.sum(onehot, axis=1, keepdims=True)
    pooled = sums / jnp.maximum(counts, 1.0)
    h3 = jnp.dot(pooled, fc1w_ref[:, :], preferred_element_type=jnp.float32)
    h3 = jnp.maximum(h3 + fc1b_ref[:, :], 0.0)
    logits = jnp.dot(h3, fc2w_ref[:, :], preferred_element_type=jnp.float32)
    logits = logits + fc2b_ref[:, :]
    z = logits - jnp.max(logits, axis=1, keepdims=True)
    e = jnp.exp(z)
    out_ref[:, :] = e / jnp.sum(e, axis=1, keepdims=True)


_stage_c = pl.pallas_call(
    _stage_c_body,
    out_shape=jax.ShapeDtypeStruct((G, D_OUT), jnp.float32),
)


@jax.jit
def kernel(x, edge_index, batch, W1, b1, g1, be1, W2, b2, g2, be2,
           fc1_w, fc1_b, fc2_w, fc2_b):
    src = edge_index[0].astype(jnp.int32)
    dst = edge_index[1].astype(jnp.int32)
    npad = EPAD - N_EDGES
    src2 = jnp.concatenate([src, jnp.zeros((npad,), jnp.int32)])
    src2 = src2.reshape(EPAD // CH, CH)
    dst2 = jnp.concatenate([dst, jnp.full((npad,), N_NODES, jnp.int32)])
    dst2 = dst2.reshape(EPAD // CH, CH)

    degp = _deg_call()(dst2).reshape(2, NPAD, 1)
    xw1, y1, dinv = _stage_a(x, W1, degp)
    acc1 = _scatter_call(SC_C0)(y1, src2, dst2)
    xw2, y2 = _stage_b(acc1, xw1, dinv, b1.reshape(1, D), g1.reshape(1, D),
                       be1.reshape(1, D), W2)
    acc2 = _scatter_call(SC_C0)(y2, src2, dst2)
    out = _stage_c(acc2, xw2, dinv, b2.reshape(1, D), g2.reshape(1, D),
                   be2.reshape(1, D), batch.reshape(1, N_NODES).astype(jnp.int32),
                   fc1_w, fc1_b.reshape(1, D), fc2_w, fc2_b.reshape(1, D_OUT))
    return out
